# Initial kernel scaffold; baseline (speedup 1.0000x reference)
import jax
import jax.numpy as jnp
from jax.experimental import pallas as pl


def kernel(pred, target):
    raise NotImplementedError("write your pallas kernel here")


# speedup vs baseline: 7.7878x; 7.7878x over previous
import functools
import math

import jax
import jax.numpy as jnp
from jax.experimental import pallas as pl

_NUM_CLASSES = 722
_BLUR_RANGE = 3
_DECAYS = [math.exp(-math.pow(2.0, d) / (2.0 * math.pow(2.0, 1))) for d in range(_BLUR_RANGE + 1)]


def _loss_kernel(target_ref, pred_ref, out_ref):
    i = pl.program_id(0)
    j = pl.program_id(1)

    p = pred_ref[0]
    tb = p.shape[0]

    m = jnp.max(p, axis=-1, keepdims=True)
    lse = m[:, 0] + jnp.log(jnp.sum(jnp.exp(p - m), axis=-1))

    tgt = target_ref[0, 0, 0, :].reshape(tb, 1)
    c = jax.lax.broadcasted_iota(jnp.int32, p.shape, 1)
    dist = jnp.abs(c - tgt)
    w = jnp.full(p.shape, 0.0, dtype=jnp.float32)
    for d in range(_BLUR_RANGE, -1, -1):
        w = jnp.where(dist == d, jnp.float32(_DECAYS[d]), w)

    wsum = jnp.sum(w, axis=-1)
    wdot = jnp.sum(w * p, axis=-1)
    partial = jnp.sum(lse * wsum - wdot).reshape(1, 1)

    @pl.when((i == 0) & (j == 0))
    def _():
        out_ref[...] = jnp.zeros_like(out_ref)

    out_ref[...] += partial


@jax.jit
def kernel(pred, target):
    B, T, C = pred.shape
    tb = 256
    nt = T // tb

    target4 = target.reshape(B, nt, 1, tb)

    out = pl.pallas_call(
        _loss_kernel,
        grid=(B, nt),
        in_specs=[
            pl.BlockSpec((1, 1, 1, tb), lambda i, j: (i, j, 0, 0)),
            pl.BlockSpec((1, tb, C), lambda i, j: (i, j, 0)),
        ],
        out_specs=pl.BlockSpec((1, 1), lambda i, j: (0, 0)),
        out_shape=jax.ShapeDtypeStruct((1, 1), jnp.float32),
    )(target4, pred)

    return out[0, 0] / (B * T)
